# baseline (speedup 1.0000x reference)
import functools

import jax
import jax.numpy as jnp
from jax import lax
from jax.experimental import pallas as pl
from jax.experimental.pallas import tpu as pltpu
from jax.experimental.pallas import tpu_sc as plsc

NUM_CORES = 2
NUM_SUBCORES = 16
NUM_WORKERS = NUM_CORES * NUM_SUBCORES
CHUNK = 128
DIM = 64
LANES = 16


@functools.lru_cache(maxsize=None)
def _make_lookup(n_chunks: int):
    mesh = plsc.VectorSubcoreMesh(core_axis_name="c", subcore_axis_name="s")

    @functools.partial(
        pl.kernel,
        mesh=mesh,
        out_type=jax.ShapeDtypeStruct((NUM_WORKERS, n_chunks, CHUNK, DIM),
                                      jnp.float32),
        scratch_types=[
            pltpu.VMEM((n_chunks, CHUNK), jnp.int32),
            pltpu.VMEM((CHUNK, DIM), jnp.float32),
            pltpu.SemaphoreType.DMA,
        ],
        compiler_params=pltpu.CompilerParams(use_tc_tiling_on_sc=False),
    )
    def lookup(table_hbm, idx_hbm, out_hbm, idx_v, rows_v, sem):
        wid = lax.axis_index("s") * NUM_CORES + lax.axis_index("c")
        pltpu.sync_copy(idx_hbm.at[wid], idx_v)

        def chunk_body(j, carry):
            pltpu.async_copy(table_hbm.at[idx_v.at[j]], rows_v, sem).wait()

            def row_body(r, c2):
                for cc in range(DIM // LANES):
                    sl = pl.ds(cc * LANES, LANES)
                    rows_v[r, sl] = rows_v[r, sl] * jnp.float32(8.0)
                return c2

            lax.fori_loop(0, CHUNK, row_body, 0)
            pltpu.sync_copy(rows_v, out_hbm.at[wid, j])
            return carry

        lax.fori_loop(0, n_chunks, chunk_body, 0)

    return lookup


def kernel(inputSWE, table):
    s, n = inputSWE.shape
    b = s * n
    n_chunks = b // (NUM_WORKERS * CHUNK)
    idx = inputSWE.reshape(NUM_WORKERS, n_chunks, CHUNK).astype(jnp.int32)
    out = _make_lookup(n_chunks)(table, idx)
    return out.reshape(s, n, DIM)


# speedup vs baseline: 1.0747x; 1.0747x over previous
import functools

import jax
import jax.numpy as jnp
from jax import lax
from jax.experimental import pallas as pl
from jax.experimental.pallas import tpu as pltpu
from jax.experimental.pallas import tpu_sc as plsc

NUM_CORES = 2
NUM_SUBCORES = 16
NUM_WORKERS = NUM_CORES * NUM_SUBCORES
CHUNK = 128
SUB = 5
BIG = SUB * CHUNK
DIM = 64
LANES = 16
SCALE = 8.0


@functools.lru_cache(maxsize=None)
def _make_lookup(n_big: int):
    mesh = plsc.VectorSubcoreMesh(core_axis_name="c", subcore_axis_name="s")
    n_idx_rows = n_big * SUB

    @functools.partial(
        pl.kernel,
        mesh=mesh,
        out_type=jax.ShapeDtypeStruct((NUM_WORKERS, n_big, BIG, DIM),
                                      jnp.float32),
        scratch_types=[
            pltpu.VMEM((n_idx_rows, CHUNK), jnp.int32),
            pltpu.VMEM((BIG, DIM), jnp.float32),
            pltpu.VMEM((BIG, DIM), jnp.float32),
            pltpu.SemaphoreType.DMA,
            pltpu.SemaphoreType.DMA,
            pltpu.SemaphoreType.DMA,
            pltpu.SemaphoreType.DMA,
        ],
        compiler_params=pltpu.CompilerParams(use_tc_tiling_on_sc=False),
    )
    def lookup(table_hbm, idx_hbm, out_hbm, idx_v, buf0, buf1,
               gs0, gs1, ss0, ss1):
        wid = lax.axis_index("s") * NUM_CORES + lax.axis_index("c")
        bufs = (buf0, buf1)
        gsems = (gs0, gs1)
        ssems = (ss0, ss1)

        pltpu.sync_copy(idx_hbm.at[wid], idx_v)

        def fire_gathers(t):
            b = t % 2
            return [
                pltpu.async_copy(
                    table_hbm.at[idx_v.at[t * SUB + k]],
                    bufs[b].at[pl.ds(k * CHUNK, CHUNK)],
                    gsems[b],
                )
                for k in range(SUB)
            ]

        pending = {0: fire_gathers(0)}
        stores = {}
        for t in range(n_big):
            b = t % 2
            if t + 1 < n_big:
                if t >= 1:
                    stores[t - 1].wait()
                pending[t + 1] = fire_gathers(t + 1)
            for c in pending[t]:
                c.wait()

            buf = bufs[b]

            @plsc.parallel_loop(0, BIG, step=1, unroll=8)
            def _scale_row(r):
                for cc in range(DIM // LANES):
                    sl = pl.ds(cc * LANES, LANES)
                    buf[r, sl] = buf[r, sl] * jnp.float32(SCALE)

            stores[t] = pltpu.async_copy(buf, out_hbm.at[wid, t], ssems[b])

        stores[n_big - 2].wait()
        stores[n_big - 1].wait()

    return lookup


def kernel(inputSWE, table):
    s, n = inputSWE.shape
    b = s * n
    n_big = b // (NUM_WORKERS * BIG)
    idx = inputSWE.reshape(NUM_WORKERS, n_big * SUB, CHUNK).astype(jnp.int32)
    out = _make_lookup(n_big)(table, idx)
    return out.reshape(s, n, DIM)
